# baseline (speedup 1.0000x reference)
import functools

import jax
import jax.numpy as jnp
from jax import lax
from jax.experimental import pallas as pl
from jax.experimental.pallas import tpu as pltpu
from jax.experimental.pallas import tpu_sc as plsc

BATCH = 16384
FIELDS = 26
DIM = 64

NC = 2
NS = 16
NW = NC * NS
ROWS_PER_W = BATCH // NW
B = BATCH * FIELDS
B_PER_W = ROWS_PER_W * FIELDS
L = 16
CHUNK = 512
N_CHUNKS = B_PER_W // CHUNK

_mesh = plsc.VectorSubcoreMesh(core_axis_name="c", subcore_axis_name="s")


@functools.partial(
    pl.kernel,
    mesh=_mesh,
    out_type=jax.ShapeDtypeStruct((B, DIM), jnp.float32),
    scratch_types=[
        pltpu.VMEM((ROWS_PER_W, FIELDS), jnp.int32),
        pltpu.VMEM((B_PER_W,), jnp.int32),
        pltpu.VMEM((CHUNK, DIM), jnp.float32),
        pltpu.SemaphoreType.DMA,
    ],
    compiler_params=pltpu.CompilerParams(
        use_tc_tiling_on_sc=False, needs_layout_passes=False
    ),
)
def _gather_kernel(emb_hbm, x_hbm, out_hbm, idx2d_v, idx_v, rows_v, sem):
    wid = lax.axis_index("s") * NC + lax.axis_index("c")
    row0 = wid * ROWS_PER_W
    base = wid * B_PER_W
    pltpu.sync_copy(x_hbm.at[pl.ds(row0, ROWS_PER_W)], idx2d_v)

    lanes = lax.iota(jnp.int32, L)

    def flatten_body(k, carry):
        r, c = carry
        v = plsc.load_gather(idx2d_v, [r, c])
        idx_v[pl.ds(k * L, L)] = v
        c = c + L
        wrap = c >= FIELDS
        c = jnp.where(wrap, c - FIELDS, c)
        r = jnp.where(wrap, r + 1, r)
        return r, c

    lax.fori_loop(
        0, B_PER_W // L, flatten_body, (jnp.zeros(L, jnp.int32), lanes)
    )

    def chunk_body(i, carry):
        off = i * CHUNK
        pltpu.async_copy(
            emb_hbm.at[idx_v.at[pl.ds(off, CHUNK)]], rows_v, sem
        ).wait()
        pltpu.sync_copy(rows_v, out_hbm.at[pl.ds(base + off, CHUNK)])
        return carry

    lax.fori_loop(0, N_CHUNKS, chunk_body, 0)


def kernel(x, emb):
    out = _gather_kernel(emb, x.astype(jnp.int32))
    return out.reshape(BATCH, FIELDS, DIM)
